# Initial kernel scaffold; baseline (speedup 1.0000x reference)
import jax
import jax.numpy as jnp
from jax.experimental import pallas as pl


def kernel(need_replace, union_feature, all_items, privacy_settings, user_sample_items, W, b):
    raise NotImplementedError("write your pallas kernel here")


# speedup vs baseline: 1.3511x; 1.3511x over previous
import functools

import jax
import jax.numpy as jnp
from jax import lax
from jax.experimental import pallas as pl
from jax.experimental.pallas import tpu as pltpu
from jax.experimental.pallas import tpu_sc as plsc

B = 16384
S = 50
D = 64

NC = 2
NS = 16
NW = NC * NS
BPW = B // NW
CH = 16
NCHUNK = BPW // CH


SP = 64
SE = 56


def _sc_gather(user_ids, user_sample_items_pad, all_items):
    mesh = plsc.VectorSubcoreMesh(core_axis_name="c", subcore_axis_name="s")

    @functools.partial(
        pl.kernel,
        mesh=mesh,
        compiler_params=pltpu.CompilerParams(use_tc_tiling_on_sc=False),
        out_type=[
            jax.ShapeDtypeStruct((B, SP), jnp.int32),
            jax.ShapeDtypeStruct((B * SE, D), jnp.float32),
        ],
        scratch_types=[
            pltpu.VMEM((BPW // 128, 128), jnp.int32),
            pltpu.VMEM((BPW, SP), jnp.int32),
            pltpu.VMEM((CH * SE, D), jnp.float32),
            pltpu.SemaphoreType.DMA,
            pltpu.SemaphoreType.DMA,
        ],
    )
    def k(uid_hbm, table_hbm, items_hbm, samp_out, emb_out,
          uid_v, samp_v, emb_v, sem, sem2):
        wid = lax.axis_index("s") * NC + lax.axis_index("c")
        base = wid * BPW
        nrow = BPW // 128
        pltpu.sync_copy(uid_hbm.at[pl.ds(wid * nrow, nrow)], uid_v)
        scopies = [
            pltpu.async_copy(
                table_hbm.at[uid_v.at[j]],
                samp_v.at[pl.ds(j * 128, 128)],
                sem2,
            )
            for j in range(nrow)
        ]
        for cp in scopies:
            cp.wait()
        pltpu.sync_copy(samp_v, samp_out.at[pl.ds(base, BPW)])

        def chunk(c, carry):
            copies = [
                pltpu.async_copy(
                    items_hbm.at[samp_v.at[c * CH + i, pl.ds(0, SE)]],
                    emb_v.at[pl.ds(i * SE, SE)],
                    sem,
                )
                for i in range(CH)
            ]
            for cp in copies:
                cp.wait()
            pltpu.sync_copy(
                emb_v, emb_out.at[pl.ds((base + c * CH) * SE, CH * SE)]
            )
            return carry

        lax.fori_loop(0, NCHUNK, chunk, None)

    return k(user_ids, user_sample_items_pad, all_items)


BB = 256


def _tc_body(e_ref, samp_ref, uf_ref, pv_ref, wt_ref, bias_ref, feat_ref, idx_ref):
    E = e_ref[...]
    u = jnp.dot(uf_ref[...], wt_ref[...][:2 * D, :],
                preferred_element_type=jnp.float32)
    u = u + pv_ref[...] * wt_ref[...][2 * D:2 * D + 1, :]
    u = u + bias_ref[...]

    parts = []
    for s in range(S):
        Es = E[:, s * D:(s + 1) * D]
        parts.append(jnp.sum(Es * u, axis=1, keepdims=True))
    scores = jnp.concatenate(parts, axis=1)
    m = jnp.max(scores, axis=1, keepdims=True)
    p = jnp.exp(scores - m)
    p = p / jnp.sum(p, axis=1, keepdims=True)

    feat = p[:, 0:1] * E[:, 0:D]
    for s in range(1, S):
        feat = feat + p[:, s:s + 1] * E[:, s * D:(s + 1) * D]
    feat_ref[...] = feat
    sampf = samp_ref[...][:, :S].astype(jnp.float32)
    idxf = jnp.sum(p * sampf, axis=1, keepdims=True)
    idx_ref[...] = idxf.astype(jnp.int32)


def _tc_compute(E2, samp, union_feature, privacy, Wt, bias):
    grid = (B // BB,)
    return pl.pallas_call(
        _tc_body,
        grid=grid,
        in_specs=[
            pl.BlockSpec((BB, SE * D), lambda i: (i, 0)),
            pl.BlockSpec((BB, SP), lambda i: (i, 0)),
            pl.BlockSpec((BB, 2 * D), lambda i: (i, 0)),
            pl.BlockSpec((BB, 1), lambda i: (i, 0)),
            pl.BlockSpec((2 * D + 1, D), lambda i: (0, 0)),
            pl.BlockSpec((1, D), lambda i: (0, 0)),
        ],
        out_specs=[
            pl.BlockSpec((BB, D), lambda i: (i, 0)),
            pl.BlockSpec((BB, 1), lambda i: (i, 0)),
        ],
        out_shape=[
            jax.ShapeDtypeStruct((B, D), jnp.float32),
            jax.ShapeDtypeStruct((B, 1), jnp.int32),
        ],
    )(E2, samp, union_feature, privacy, Wt, bias)


def kernel(need_replace, union_feature, all_items, privacy_settings, user_sample_items, W, b):
    user_ids = need_replace[:, 0].reshape(B // 128, 128)
    ust_pad = jnp.pad(user_sample_items, ((0, 0), (0, SP - S)))
    samp, emb = _sc_gather(user_ids, ust_pad, all_items)
    E2 = emb.reshape(B, SE * D)
    feat, idx = _tc_compute(
        E2, samp, union_feature,
        privacy_settings.reshape(B, 1), W.T, b.reshape(1, D),
    )
    return (idx.reshape(B), feat, 0.0, 0.0)
